# baseline (speedup 1.0000x reference)
import functools

import jax
import jax.numpy as jnp
import numpy as np
from jax import lax
from jax.experimental import pallas as pl
from jax.experimental.pallas import tpu as pltpu
from jax.experimental.pallas import tpu_sc as plsc

N = 50000
E = 400000
DIN = 128
DH = 64
H = 4
D = 16
L = 2

NC = 2
NS = 16
CHB = 128
KS = 5
CH = CHB * KS
NBLK = E // CH
NBLK2 = E // CHB
NHALF = N // NC
UPAD = 25024
TPT = UPAD // NS
BN_TC = 2000
BN_F = 1000


def _inproj_body(x_ref, w_ref, b_ref, o_ref):
    o_ref[...] = jnp.dot(x_ref[...], w_ref[...],
                         preferred_element_type=jnp.float32) + b_ref[...]


def _input_proj(x, w, b):
    return pl.pallas_call(
        _inproj_body,
        grid=(N // BN_TC,),
        in_specs=[
            pl.BlockSpec((BN_TC, DIN), lambda i: (i, 0)),
            pl.BlockSpec((DIN, DH), lambda i: (0, 0)),
            pl.BlockSpec((1, DH), lambda i: (0, 0)),
        ],
        out_specs=pl.BlockSpec((BN_TC, DH), lambda i: (i, 0)),
        out_shape=jax.ShapeDtypeStruct((N, DH), jnp.float32),
    )(x, w, b)


def _proj_body(x_ref, wq, bq, qs, wk, bk, a, wv, bv, mm, q_ref, k_ref, v_ref):
    x = x_ref[...]
    f32 = jnp.float32
    q_ref[...] = (jnp.dot(x, wq[...], preferred_element_type=f32)
                  + bq[...]) * qs[...]
    k_ref[...] = jnp.dot(jnp.dot(x, wk[...], preferred_element_type=f32)
                         + bk[...], a[...], preferred_element_type=f32)
    v_ref[...] = jnp.dot(jnp.dot(x, wv[...], preferred_element_type=f32)
                         + bv[...], mm[...], preferred_element_type=f32)


def _layer_proj(x, wq, bq, qs, wk, bk, a, wv, bv, mm):
    wspec = pl.BlockSpec((DH, DH), lambda i: (0, 0))
    bspec = pl.BlockSpec((1, DH), lambda i: (0, 0))
    return pl.pallas_call(
        _proj_body,
        grid=(N // BN_TC,),
        in_specs=[pl.BlockSpec((BN_TC, DH), lambda i: (i, 0)),
                  wspec, bspec, bspec, wspec, bspec, wspec, wspec, bspec, wspec],
        out_specs=[pl.BlockSpec((BN_TC, DH), lambda i: (i, 0))] * 3,
        out_shape=[jax.ShapeDtypeStruct((N, DH), jnp.float32)] * 3,
    )(x, wq, bq, qs, wk, bk, a, wv, bv, mm)


def _final_body(u_ref, s_ref, x_ref, p8_ref, wa, ba_r, skp, o_ref, *, do_elu):
    f32 = jnp.float32
    u = u_ref[0]
    sx = jnp.dot(s_ref[0], p8_ref[...], preferred_element_type=f32)
    agg = u / (sx + 1e-30)
    o = jnp.dot(jax.nn.gelu(agg), wa[...], preferred_element_type=f32) + ba_r[...]
    a_s = jax.nn.sigmoid(skp[0, 0])
    o = a_s * o + (1.0 - a_s) * x_ref[...]
    if do_elu:
        o = jnp.where(o > 0, o, jnp.exp(jnp.minimum(o, 0.0)) - 1.0)
    o_ref[...] = o


def _final_stage(u, s, x, p8, wa, ba_r, skp, do_elu):
    nb = NHALF // BN_F
    return pl.pallas_call(
        functools.partial(_final_body, do_elu=do_elu),
        grid=(N // BN_F,),
        in_specs=[
            pl.BlockSpec((1, BN_F, DH), lambda i, nb=nb: (i // nb, i % nb, 0)),
            pl.BlockSpec((1, BN_F, 4), lambda i, nb=nb: (i // nb, i % nb, 0)),
            pl.BlockSpec((BN_F, DH), lambda i: (i, 0)),
            pl.BlockSpec((4, DH), lambda i: (0, 0)),
            pl.BlockSpec((DH, DH), lambda i: (0, 0)),
            pl.BlockSpec((1, DH), lambda i: (0, 0)),
            pl.BlockSpec((1, 1), lambda i: (0, 0)),
        ],
        out_specs=pl.BlockSpec((BN_F, DH), lambda i: (i, 0)),
        out_shape=jax.ShapeDtypeStruct((N, DH), jnp.float32),
    )(u, s, x, p8, wa, ba_r, skp)


@functools.lru_cache(maxsize=None)
def _sc_mesh():
    return plsc.VectorSubcoreMesh(core_axis_name="c", subcore_axis_name="s",
                                  num_cores=NC, num_subcores=NS)


_GDN = lax.GatherDimensionNumbers(offset_dims=(), collapsed_slice_dims=(0,),
                                  start_index_map=(0,))


def _splat_lane(vec, lane):
    idx = jnp.full((16, 1), lane, jnp.int32)
    return lax.gather(vec, idx, dimension_numbers=_GDN, slice_sizes=(1,),
                      mode=lax.GatherScatterMode.PROMISE_IN_BOUNDS)


def _pass1_body(qd0, k0, si0, di0, qd1, k1, si1, di1,
                lg0, lg1, mx0, mx1,
                sib, dib, qb, kb, lb, mxb, gsem):
    c = lax.axis_index("c")
    s = lax.axis_index("s")
    w = c * NS + s
    iot = lax.iota(jnp.int32, 16)
    rowp = lax.div(iot, 4)
    colp = (iot % 4) * 16
    for et in range(2):
        qd, ke, si, di, lg, mx = ((qd0, k0, si0, di0, lg0, mx0),
                                  (qd1, k1, si1, di1, lg1, mx1))[et]
        mxb[...] = jnp.full((16,), -jnp.inf, jnp.float32)

        def blk(t, carry, qd=qd, ke=ke, si=si, di=di, lg=lg):
            b = w + (NC * NS) * t

            @pl.when(b < NBLK)
            def _():
                base = b * CH
                for k in range(KS):
                    pltpu.sync_copy(si.at[pl.ds(base + k * CHB, CHB)], sib.at[k])
                    pltpu.sync_copy(di.at[pl.ds(base + k * CHB, CHB)], dib.at[k])
                cps = []
                for k in range(KS):
                    cps.append(pltpu.async_copy(
                        qd.at[dib.at[k]], qb.at[pl.ds(k * CHB, CHB)], gsem))
                    cps.append(pltpu.async_copy(
                        ke.at[sib.at[k]], kb.at[pl.ds(k * CHB, CHB)], gsem))
                for cp in cps:
                    cp.wait()

                def sub(qq, carry2):
                    off = qq * CHB
                    rm = mxb[...]
                    for g in range(CHB // 4):
                        rowv = rowp + (off + 4 * g)
                        acc = jnp.zeros((16,), jnp.float32)
                        for dd in range(16):
                            colv = colp + dd
                            acc = acc + (plsc.load_gather(qb, [rowv, colv])
                                         * plsc.load_gather(kb, [rowv, colv]))
                        lb[pl.ds(off * H + g * 16, 16)] = acc
                        rm = jnp.maximum(rm, acc)
                    mxb[...] = rm
                    return carry2

                lax.fori_loop(0, KS, sub, 0)
                pltpu.sync_copy(lb, lg.at[pl.ds(base * H, CH * H)])
            return carry

        lax.fori_loop(0, (NBLK + NC * NS - 1) // (NC * NS), blk, 0)
        pltpu.sync_copy(mxb, mx.at[w])


def _sc_pass1(qd0, k0, si0, di0, qd1, k1, si1, di1):
    f = pl.kernel(
        _pass1_body,
        out_type=(jax.ShapeDtypeStruct((E * H,), jnp.float32),
                  jax.ShapeDtypeStruct((E * H,), jnp.float32),
                  jax.ShapeDtypeStruct((NC * NS, 16), jnp.float32),
                  jax.ShapeDtypeStruct((NC * NS, 16), jnp.float32)),
        mesh=_sc_mesh(),
        compiler_params=pltpu.CompilerParams(needs_layout_passes=False, use_tc_tiling_on_sc=False),
        scratch_types=[
            pltpu.VMEM((KS, CHB), jnp.int32),
            pltpu.VMEM((KS, CHB), jnp.int32),
            pltpu.VMEM((CH, DH), jnp.float32),
            pltpu.VMEM((CH, DH), jnp.float32),
            pltpu.VMEM((CH * H,), jnp.float32),
            pltpu.VMEM((16,), jnp.float32),
            pltpu.SemaphoreType.DMA,
        ],
    )
    return f(qd0, k0, si0, di0, qd1, k1, si1, di1)


def _pass2_body(lg0, mx0, v0, si0, di0, lg1, mx1, v1, si1, di1, zU, zS,
                u0, s0o, u1, s1o,
                sib, dib, didxb, vb, wb, whb, sidx, lb, mxb2,
                isem0, isem1, gsem0, gsem1, ush, ssh):
    c = lax.axis_index("c")
    s = lax.axis_index("s")
    iot = lax.iota(jnp.int32, 16)
    base_half = c * NHALF
    r0 = s * TPT
    isems = (isem0, isem1)
    gsems = (gsem0, gsem1)

    def fire_idx(sl, b, si, di):
        @pl.when(b < NBLK2)
        def _():
            pltpu.async_copy(si.at[pl.ds(b * CHB, CHB)], sib.at[sl], isems[sl])
            pltpu.async_copy(di.at[pl.ds(b * CHB, CHB)], dib.at[sl], isems[sl])

    def fire_gather(sl, b, si, ve, lg):
        @pl.when(b < NBLK2)
        def _():
            pltpu.make_async_copy(si.at[pl.ds(0, CHB)], sib.at[sl],
                                  isems[sl]).wait()
            pltpu.make_async_copy(si.at[pl.ds(0, CHB)], dib.at[sl],
                                  isems[sl]).wait()
            pltpu.async_copy(ve.at[sib.at[sl]], vb.at[sl], gsems[sl])
            pltpu.async_copy(lg.at[pl.ds(b * CHB * H, CHB * H)], lb.at[sl],
                             gsems[sl])

    def wait_didx(sl, b, ve, lg):
        @pl.when(b < NBLK2)
        def _():
            pltpu.make_async_copy(ve.at[sib.at[sl]], vb.at[sl],
                                  gsems[sl]).wait()
            pltpu.make_async_copy(lg.at[pl.ds(0, CHB * H)], lb.at[sl],
                                  gsems[sl]).wait()
            for j in range(CHB // 16):
                col = j * 16
                dv = dib[sl, pl.ds(col, 16)]
                hv = dv - base_half
                ok = (hv >= 0) & (hv < NHALF)
                dd = jnp.where(ok, hv, NHALF + jnp.bitwise_and(dv, 7))
                didxb[0, pl.ds(col, 16)] = dd
                for h in range(H):
                    sidx[h, pl.ds(col, 16)] = dd * H + h

    def process(sl, b, gmax):
        @pl.when(b < NBLK2)
        def _():
            for g in range(CHB // 4):
                fl = g * 16
                wv = jnp.exp(lb[sl, pl.ds(fl, 16)] - gmax)
                wb[pl.ds(fl, 16)] = wv
                for e4 in range(4):
                    e = g * 4 + e4
                    for h in range(H):
                        bw = _splat_lane(wv, e4 * 4 + h)
                        vb[sl, e, pl.ds(h * 16, 16)] = (
                            bw * vb[sl, e, pl.ds(h * 16, 16)])
            for j in range(CHB // 16):
                for h in range(H):
                    wh = plsc.load_gather(wb, [j * 64 + iot * H + h])
                    whb[h, pl.ds(j * 16, 16)] = wh
            pltpu.sync_copy(vb.at[sl], ush.at[didxb.at[0]], add=True)
            for h in range(H):
                pltpu.sync_copy(whb.at[h], ssh.at[sidx.at[h]], add=True)

    for et in range(2):
        lg, mx, ve, si, di, uo, so = ((lg0, mx0, v0, si0, di0, u0, s0o),
                                      (lg1, mx1, v1, si1, di1, u1, s1o))[et]
        pltpu.sync_copy(zU.at[pl.ds(r0, TPT)], ush.at[pl.ds(r0, TPT)])
        pltpu.sync_copy(zS.at[pl.ds(r0 * H, TPT * H)], ssh.at[pl.ds(r0 * H, TPT * H)])
        pltpu.sync_copy(mx, mxb2)
        acc = mxb2[0]
        for i in range(1, NC * NS):
            acc = jnp.maximum(acc, mxb2[i])
        gmax = jnp.max(acc)
        plsc.subcore_barrier()

        fire_idx(0, s, si, di)
        fire_idx(1, s + NS, si, di)
        fire_gather(0, s, si, ve, lg)

        def pair(tt, carry, lg=lg, ve=ve, si=si, di=di, gmax=gmax):
            b0 = s + NS * (2 * tt)
            b1 = s + NS * (2 * tt + 1)
            b2 = s + NS * (2 * tt + 2)
            b3 = s + NS * (2 * tt + 3)
            wait_didx(0, b0, ve, lg)
            fire_idx(0, b2, si, di)
            fire_gather(1, b1, si, ve, lg)
            process(0, b0, gmax)
            wait_didx(1, b1, ve, lg)
            fire_idx(1, b3, si, di)
            fire_gather(0, b2, si, ve, lg)
            process(1, b1, gmax)
            return carry

        lax.fori_loop(0, (NBLK2 + 2 * NS - 1) // (2 * NS), pair, 0)
        plsc.subcore_barrier()
        pltpu.sync_copy(ush.at[pl.ds(r0, TPT)], uo.at[c, pl.ds(r0, TPT)])
        pltpu.sync_copy(ssh.at[pl.ds(r0 * H, TPT * H)], so.at[c, pl.ds(r0 * H, TPT * H)])
        plsc.subcore_barrier()


def _sc_pass2(lg0, mx0, v0, si0, di0, lg1, mx1, v1, si1, di1, zU, zS):
    f = pl.kernel(
        _pass2_body,
        out_type=(jax.ShapeDtypeStruct((NC, UPAD, DH), jnp.float32),
                  jax.ShapeDtypeStruct((NC, UPAD * H), jnp.float32),
                  jax.ShapeDtypeStruct((NC, UPAD, DH), jnp.float32),
                  jax.ShapeDtypeStruct((NC, UPAD * H), jnp.float32)),
        mesh=_sc_mesh(),
        compiler_params=pltpu.CompilerParams(needs_layout_passes=False, use_tc_tiling_on_sc=False),
        scratch_types=[
            pltpu.VMEM((2, CHB), jnp.int32),
            pltpu.VMEM((2, CHB), jnp.int32),
            pltpu.VMEM((1, CHB), jnp.int32),
            pltpu.VMEM((2, CHB, DH), jnp.float32),
            pltpu.VMEM((CHB * H,), jnp.float32),
            pltpu.VMEM((H, CHB), jnp.float32),
            pltpu.VMEM((H, CHB), jnp.int32),
            pltpu.VMEM((2, CHB * H), jnp.float32),
            pltpu.VMEM((NC * NS, 16), jnp.float32),
            pltpu.SemaphoreType.DMA,
            pltpu.SemaphoreType.DMA,
            pltpu.SemaphoreType.DMA,
            pltpu.SemaphoreType.DMA,
            pltpu.VMEM_SHARED((UPAD, DH), jnp.float32),
            pltpu.VMEM_SHARED((UPAD * H,), jnp.float32),
        ],
    )
    return f(lg0, mx0, v0, si0, di0, lg1, mx1, v1, si1, di1, zU, zS)


def _blockdiag(blocks):
    return jax.scipy.linalg.block_diag(*[blocks[h] for h in range(H)])


def kernel(x_user, x_item, ei_u2i, ei_i2u, Win, b_in, Wk, bk, Wq, bq, Wv, bv,
           Wa, ba, skip, a_rel, m_rel, p_rel):
    si0 = ei_u2i[0].astype(jnp.int32)
    di0 = ei_u2i[1].astype(jnp.int32)
    si1 = ei_i2u[0].astype(jnp.int32)
    di1 = ei_i2u[1].astype(jnp.int32)
    zU = jnp.zeros((UPAD, DH), jnp.float32)
    zS = jnp.zeros((UPAD * H,), jnp.float32)
    p8 = jnp.zeros((H, DH), jnp.float32)
    p8 = p8.at[np.arange(H).repeat(D), np.arange(DH)].set(1.0)

    X = [_input_proj(x_user, Win[0], b_in[0].reshape(1, DH)),
         _input_proj(x_item, Win[1], b_in[1].reshape(1, DH))]
    for l in range(L):
        Q, K, V = [], [], []
        for nt in range(2):
            qs = (jnp.repeat(p_rel[l, 1 - nt], D) / np.sqrt(D)).reshape(1, DH)
            q, k, v = _layer_proj(
                X[nt], Wq[l, nt], bq[l, nt].reshape(1, DH), qs,
                Wk[l, nt], bk[l, nt].reshape(1, DH), _blockdiag(a_rel[l, nt]),
                Wv[l, nt], bv[l, nt].reshape(1, DH), _blockdiag(m_rel[l, nt]))
            Q.append(q); K.append(k); V.append(v)
        lg0, lg1, mx0, mx1 = _sc_pass1(Q[1], K[0], si0, di0,
                                       Q[0], K[1], si1, di1)
        u0, s0, u1, s1 = _sc_pass2(lg0, mx0, V[0], si0, di0,
                                   lg1, mx1, V[1], si1, di1, zU, zS)
        newX = []
        for nt in range(2):
            u, sden = (u1, s1) if nt == 0 else (u0, s0)
            newX.append(_final_stage(
                u, sden.reshape(NC, UPAD, H), X[nt], p8, Wa[l, nt], ba[l, nt].reshape(1, DH),
                skip[l, nt].reshape(1, 1), do_elu=(l < L - 1)))
        X = newX
    return jnp.stack(X, axis=0)
